# baseline (speedup 1.0000x reference)
import functools

import jax
import jax.numpy as jnp
from jax import lax
from jax.experimental import pallas as pl
from jax.experimental.pallas import tpu as pltpu
from jax.experimental.pallas import tpu_sc as plsc

BLANK_LABEL = 10
NUM_FRAMES = 8192
NUM_LABELS = 32
NUM_WORKERS = 16
F_PER = NUM_FRAMES // NUM_WORKERS
GROUPS = F_PER // 16
WORDS_PER = F_PER * NUM_LABELS


def _ctc_body(emis_hbm, out_hbm, cnt_hbm, stage_hbm,
              emis_v, idx_v, loc_v, dat_v, out_v, tmp_v, off_v,
              cnt_s, sem, sem2):
    wid = lax.axis_index("s")
    io = lax.iota(jnp.int32, 16)
    io32 = io * NUM_LABELS

    base = wid * WORDS_PER
    half = WORDS_PER // 2
    cp1 = pltpu.async_copy(emis_hbm.at[pl.ds(base, half)],
                           emis_v.at[pl.ds(NUM_LABELS, half)], sem)
    cp2 = pltpu.async_copy(emis_hbm.at[pl.ds(base + half, half)],
                           emis_v.at[pl.ds(NUM_LABELS + half, half)], sem2)

    @pl.when(wid > 0)
    def _():
        pltpu.sync_copy(emis_hbm.at[pl.ds(base - NUM_LABELS, NUM_LABELS)],
                        emis_v.at[pl.ds(0, NUM_LABELS)])

    for w in range(NUM_WORKERS):
        cnt_s[w] = jnp.int32(0)
    plsc.subcore_barrier()
    cp1.wait()

    v0 = emis_v[pl.ds(0, 16)]
    v1 = emis_v[pl.ds(16, 16)]
    mx = jnp.max(jnp.maximum(v0, v1))
    c0 = jnp.where(v0 == mx, io, 64)
    c1 = jnp.where(v1 == mx, io + 16, 64)
    p = jnp.min(jnp.minimum(c0, c1))
    p = jnp.where(wid == 0, -1, p)
    idx_v[pl.ds(0, 16)] = jnp.broadcast_to(p, (16,))

    def amax_group(g):
        b0 = io32 + (g * (16 * NUM_LABELS) + NUM_LABELS)
        cj = io
        cm = plsc.load_gather(emis_v, [b0 + io])
        ci = cj
        for j in range(1, NUM_LABELS):
            cj = (cj + 1) & (NUM_LABELS - 1)
            vj = plsc.load_gather(emis_v, [b0 + cj])
            gt = vj > cm
            tie = (vj == cm) & (cj < ci)
            upd = gt | tie
            cm = jnp.where(upd, vj, cm)
            ci = jnp.where(upd, cj, ci)
        idx_v[pl.ds(16 + g * 16, 16)] = ci

    @plsc.parallel_loop(0, GROUPS // 2, unroll=1)
    def _amax_lo(g):
        amax_group(g)

    cp2.wait()

    @plsc.parallel_loop(GROUPS // 2, GROUPS, unroll=1)
    def _amax_hi(g):
        amax_group(g)

    def comp_body(g, off):
        cur = idx_v[pl.ds(16 + g * 16, 16)]
        prev = plsc.load_gather(idx_v, [io + (g * 16 + 15)])
        keep = (cur != prev) & (cur != BLANK_LABEL)
        k32 = keep.astype(jnp.int32)
        pos = off + plsc.cumsum(k32) - 1
        plsc.store_scatter(loc_v, [pos], cur, mask=keep)
        return off + jnp.sum(k32)

    cnt = lax.fori_loop(0, GROUPS, comp_body, jnp.int32(0))

    pltpu.sync_copy(loc_v, stage_hbm.at[pl.ds(wid * F_PER, F_PER)])

    def push_body(j, carry):
        plsc.fetch_and_add(cnt_s.at[wid], cnt, subcore_id=j)
        return carry

    lax.fori_loop(0, NUM_WORKERS, push_body, 0)
    plsc.subcore_barrier()

    cvec = jnp.zeros((16,), jnp.int32)
    for w in range(NUM_WORKERS):
        cvec = jnp.where(io == w, cnt_s[w], cvec)
    offs = plsc.cumsum(cvec) - cvec
    total = jnp.sum(cvec)
    off_v[pl.ds(0, 16)] = offs

    @pl.when(wid == 0)
    def _():
        tmp_v[pl.ds(0, 16)] = jnp.broadcast_to(total, (16,))
        pltpu.sync_copy(tmp_v, cnt_hbm)

    pltpu.sync_copy(stage_hbm, dat_v)

    off_sc = [jnp.int32(0)]
    for w in range(1, NUM_WORKERS):
        off_sc.append(off_sc[w - 1] + cnt_s[w - 1])
    obc = [jnp.broadcast_to(off_sc[w], (16,)) for w in range(1, NUM_WORKERS)]
    tbc = jnp.broadcast_to(total, (16,))

    @plsc.parallel_loop(0, GROUPS, unroll=2)
    def _emit(t):
        k = wid * F_PER + t * 16 + io
        s = jnp.zeros((16,), jnp.int32)
        for w in range(1, NUM_WORKERS):
            s = s + (k >= obc[w - 1]).astype(jnp.int32)
        osrc = plsc.load_gather(off_v, [s])
        src = jnp.minimum((s * F_PER) + k - osrc, NUM_FRAMES - 1)
        val = plsc.load_gather(dat_v, [src])
        out_v[pl.ds(t * 16, 16)] = jnp.where(k < tbc, val, -1)

    pltpu.sync_copy(out_v, out_hbm.at[pl.ds(wid * F_PER, F_PER)])


_ctc_call = functools.partial(
    pl.kernel,
    out_type=[jax.ShapeDtypeStruct((NUM_FRAMES,), jnp.int32),
              jax.ShapeDtypeStruct((16,), jnp.int32),
              jax.ShapeDtypeStruct((NUM_FRAMES,), jnp.int32)],
    mesh=plsc.VectorSubcoreMesh(core_axis_name="c", subcore_axis_name="s",
                                num_cores=1),
    compiler_params=pltpu.CompilerParams(needs_layout_passes=False),
    scratch_types=[
        pltpu.VMEM(((F_PER + 1) * NUM_LABELS,), jnp.float32),
        pltpu.VMEM((16 + F_PER,), jnp.int32),
        pltpu.VMEM((F_PER,), jnp.int32),
        pltpu.VMEM((NUM_FRAMES,), jnp.int32),
        pltpu.VMEM((F_PER,), jnp.int32),
        pltpu.VMEM((16,), jnp.int32),
        pltpu.VMEM((16,), jnp.int32),
        pltpu.SMEM((NUM_WORKERS,), jnp.int32),
        pltpu.SemaphoreType.DMA,
        pltpu.SemaphoreType.DMA,
    ],
)(_ctc_body)


def kernel(emission):
    compacted, counts, _ = _ctc_call(emission.reshape(-1))
    return compacted, counts[0]


# speedup vs baseline: 1.0484x; 1.0484x over previous
import functools

import jax
import jax.numpy as jnp
from jax import lax
from jax.experimental import pallas as pl
from jax.experimental.pallas import tpu as pltpu
from jax.experimental.pallas import tpu_sc as plsc

BLANK_LABEL = 10
NUM_FRAMES = 8192
NUM_LABELS = 32
NUM_WORKERS = 16
F_PER = NUM_FRAMES // NUM_WORKERS
GROUPS = F_PER // 16
WORDS_PER = F_PER * NUM_LABELS


def _ctc_body(emis_hbm, out_hbm, cnt_hbm, stage_hbm,
              emis_v, idx_v, loc_v, dat_v, out_v, tmp_v, off_v,
              cnt_s, sem):
    wid = lax.axis_index("s")
    io = lax.iota(jnp.int32, 16)
    io32 = io * NUM_LABELS

    base = wid * WORDS_PER
    cp = pltpu.async_copy(emis_hbm.at[pl.ds(base, WORDS_PER)],
                          emis_v.at[pl.ds(NUM_LABELS, WORDS_PER)], sem)

    @pl.when(wid > 0)
    def _():
        pltpu.sync_copy(emis_hbm.at[pl.ds(base - NUM_LABELS, NUM_LABELS)],
                        emis_v.at[pl.ds(0, NUM_LABELS)])

    for w in range(NUM_WORKERS):
        cnt_s[w] = jnp.int32(0)
    plsc.subcore_barrier()
    cp.wait()

    v0 = emis_v[pl.ds(0, 16)]
    v1 = emis_v[pl.ds(16, 16)]
    mx = jnp.max(jnp.maximum(v0, v1))
    c0 = jnp.where(v0 == mx, io, 64)
    c1 = jnp.where(v1 == mx, io + 16, 64)
    p = jnp.min(jnp.minimum(c0, c1))
    p = jnp.where(wid == 0, -1, p)
    idx_v[pl.ds(0, 16)] = jnp.broadcast_to(p, (16,))

    @plsc.parallel_loop(0, GROUPS, unroll=2)
    def _amax(g):
        b0 = io32 + (g * (16 * NUM_LABELS) + NUM_LABELS)
        cj = io
        cm = plsc.load_gather(emis_v, [b0 + io])
        ci = cj
        for j in range(1, NUM_LABELS):
            cj = (cj + 1) & (NUM_LABELS - 1)
            vj = plsc.load_gather(emis_v, [b0 + cj])
            gt = vj > cm
            tie = (vj == cm) & (cj < ci)
            upd = gt | tie
            cm = jnp.where(upd, vj, cm)
            ci = jnp.where(upd, cj, ci)
        idx_v[pl.ds(16 + g * 16, 16)] = ci

    def comp_body(g, off):
        cur = idx_v[pl.ds(16 + g * 16, 16)]
        prev = plsc.load_gather(idx_v, [io + (g * 16 + 15)])
        keep = (cur != prev) & (cur != BLANK_LABEL)
        k32 = keep.astype(jnp.int32)
        pos = off + plsc.cumsum(k32) - 1
        plsc.store_scatter(loc_v, [pos], cur, mask=keep)
        return off + jnp.sum(k32)

    cnt = lax.fori_loop(0, GROUPS, comp_body, jnp.int32(0))

    pltpu.sync_copy(loc_v, stage_hbm.at[pl.ds(wid * F_PER, F_PER)])

    def push_body(j, carry):
        plsc.fetch_and_add(cnt_s.at[wid], cnt, subcore_id=j)
        return carry

    lax.fori_loop(0, NUM_WORKERS, push_body, 0)
    plsc.subcore_barrier()

    cvec = jnp.zeros((16,), jnp.int32)
    for w in range(NUM_WORKERS):
        cvec = jnp.where(io == w, cnt_s[w], cvec)
    offs = plsc.cumsum(cvec) - cvec
    total = jnp.sum(cvec)
    off_v[pl.ds(0, 16)] = offs

    @pl.when(wid == 0)
    def _():
        tmp_v[pl.ds(0, 16)] = jnp.broadcast_to(total, (16,))
        pltpu.sync_copy(tmp_v, cnt_hbm)

    pltpu.sync_copy(stage_hbm, dat_v)

    off_sc = [jnp.int32(0)]
    for w in range(1, NUM_WORKERS):
        off_sc.append(off_sc[w - 1] + cnt_s[w - 1])
    obc = [jnp.broadcast_to(off_sc[w], (16,)) for w in range(1, NUM_WORKERS)]
    tbc = jnp.broadcast_to(total, (16,))

    @plsc.parallel_loop(0, GROUPS, unroll=2)
    def _emit(t):
        k = wid * F_PER + t * 16 + io
        s = jnp.zeros((16,), jnp.int32)
        for w in range(1, NUM_WORKERS):
            s = s + (k >= obc[w - 1]).astype(jnp.int32)
        osrc = plsc.load_gather(off_v, [s])
        src = jnp.minimum((s * F_PER) + k - osrc, NUM_FRAMES - 1)
        val = plsc.load_gather(dat_v, [src])
        out_v[pl.ds(t * 16, 16)] = jnp.where(k < tbc, val, -1)

    pltpu.sync_copy(out_v, out_hbm.at[pl.ds(wid * F_PER, F_PER)])


_ctc_call = functools.partial(
    pl.kernel,
    out_type=[jax.ShapeDtypeStruct((NUM_FRAMES,), jnp.int32),
              jax.ShapeDtypeStruct((16,), jnp.int32),
              jax.ShapeDtypeStruct((NUM_FRAMES,), jnp.int32)],
    mesh=plsc.VectorSubcoreMesh(core_axis_name="c", subcore_axis_name="s",
                                num_cores=1),
    compiler_params=pltpu.CompilerParams(needs_layout_passes=False),
    scratch_types=[
        pltpu.VMEM(((F_PER + 1) * NUM_LABELS,), jnp.float32),
        pltpu.VMEM((16 + F_PER,), jnp.int32),
        pltpu.VMEM((F_PER,), jnp.int32),
        pltpu.VMEM((NUM_FRAMES,), jnp.int32),
        pltpu.VMEM((F_PER,), jnp.int32),
        pltpu.VMEM((16,), jnp.int32),
        pltpu.VMEM((16,), jnp.int32),
        pltpu.SMEM((NUM_WORKERS,), jnp.int32),
        pltpu.SemaphoreType.DMA,
    ],
)(_ctc_body)


def kernel(emission):
    compacted, counts, _ = _ctc_call(emission.reshape(-1))
    return compacted, counts[0]
